# baseline (speedup 1.0000x reference)
import functools

import jax
import jax.numpy as jnp
from jax import lax
from jax.experimental import pallas as pl
from jax.experimental.pallas import tpu as pltpu
from jax.experimental.pallas import tpu_sc as plsc

_INFO = plsc.get_sparse_core_info()
_NC = _INFO.num_cores
_NS = _INFO.num_subcores
_NW = _NC * _NS
_G = 128
_CPW = 4
_BPW = _CPW * _G


def _make_gather(batch, hist, vocab, dim):
    assert batch == _NW * _BPW and dim == 32
    n_rt = dim // 8
    mesh = plsc.VectorSubcoreMesh(core_axis_name="c", subcore_axis_name="s")

    @functools.partial(
        pl.kernel,
        out_type=jax.ShapeDtypeStruct((hist, n_rt, batch // _G, 8 * _G), jnp.float32),
        mesh=mesh,
        scratch_types=[
            pltpu.VMEM((2, _BPW), jnp.int32),
            pltpu.VMEM((2, _BPW, dim), jnp.float32),
            pltpu.VMEM((2, _CPW, n_rt, 8 * _G), jnp.float32),
            pltpu.VMEM((16, 16), jnp.int32),
            pltpu.VMEM((16, 16), jnp.int32),
            pltpu.SemaphoreType.DMA,
            pltpu.SemaphoreType.DMA,
            pltpu.SemaphoreType.DMA,
        ],
        compiler_params=pltpu.CompilerParams(
            use_tc_tiling_on_sc=False,
            needs_layout_passes=False,
            disable_bounds_checks=True,
        ),
    )
    def gather_kernel(idx_hbm, table_hbm, out_hbm, idx_v, rows_v, tv, perm_v,
                      pofs_v, isem, gsem, osem):
        c = lax.axis_index("c")
        s = lax.axis_index("s")
        wid = s * _NC + c
        b0 = wid * _BPW

        iota = lax.iota(jnp.int32, 16)
        for sh in range(16):
            p = lax.rem(iota + sh, 16)
            perm_v[sh] = p
            pofs_v[sh] = lax.rem(iota, 8) * 128 + p
        rowsel = lax.shift_right_logical(iota, 3)
        rowsel_dh = [rowsel, rowsel + 2]
        col_dh = [iota, iota + 16]

        def wait_idx(buf):
            pltpu.make_async_copy(
                idx_hbm.at[0, pl.ds(0, _BPW)], idx_v.at[buf], isem
            ).wait()

        def wait_rows(buf):
            pltpu.make_async_copy(
                table_hbm.at[pl.ds(0, _BPW)], rows_v.at[buf], gsem
            ).wait()

        def wait_out(buf):
            pltpu.make_async_copy(
                tv.at[buf], out_hbm.at[0, pl.ds(0, n_rt), pl.ds(0, _CPW)], osem
            ).wait()

        def fire_idx(h, buf):
            pltpu.async_copy(idx_hbm.at[h, pl.ds(b0, _BPW)], idx_v.at[buf], isem)

        def fire_gathers(buf):
            for j in range(_CPW):
                pltpu.async_copy(
                    table_hbm.at[idx_v.at[buf, pl.ds(j * _G, _G)]],
                    rows_v.at[buf, pl.ds(j * _G, _G)],
                    gsem,
                )

        def transpose(buf):
            def s_body(sq, carry):
                for si in range(8):
                    sh = 8 * sq + si
                    perm = perm_v[sh]
                    pofs = pofs_v[sh]
                    for j in range(8):
                        pofs_j = pofs + (j * 16)
                        for cc in range(_CPW):
                            src_row = perm + (cc * _G + j * 16)
                            for dh in range(2):
                                vals = plsc.load_gather(
                                    rows_v.at[buf], [src_row, col_dh[dh]]
                                )
                                plsc.store_scatter(
                                    tv.at[buf, cc],
                                    [rowsel_dh[dh], pofs_j],
                                    vals,
                                )
                return carry

            lax.fori_loop(0, 2, s_body, 0)

        def fire_writes(h, buf):
            for cc in range(_CPW):
                pltpu.async_copy(
                    tv.at[buf, cc],
                    out_hbm.at[h, pl.ds(0, n_rt), wid * _CPW + cc],
                    osem,
                )

        fire_idx(0, 0)
        wait_idx(0)
        fire_gathers(0)
        fire_idx(1, 1)

        def step(h, b):
            ob = 1 - b
            wait_rows(ob)

            @pl.when(h + 1 <= hist - 1)
            def _():
                fire_idx(h + 1, ob)

            @pl.when(h <= hist - 1)
            def _():
                wait_idx(b)
                fire_gathers(b)

            @pl.when(h >= 3)
            def _():
                wait_out(ob)

            transpose(ob)
            fire_writes(h - 1, ob)

        def pair(t, carry):
            step(2 * t + 1, 1)
            step(2 * t + 2, 0)
            return carry

        lax.fori_loop(0, hist // 2, pair, 0)

        wait_out(0)
        wait_out(1)

    return gather_kernel


def kernel(input_tokens, table):
    batch, hist = input_tokens.shape
    vocab, dim = table.shape
    idx_t = input_tokens.T.astype(jnp.int32)
    out5 = _make_gather(batch, hist, vocab, dim)(idx_t, table)
    out = out5.reshape(hist, dim // 8, batch // _G, 8, _G)
    out = out.transpose(2, 4, 0, 1, 3)
    return out.reshape(batch, hist, dim)


# speedup vs baseline: 1.6157x; 1.6157x over previous
import functools

import jax
import jax.numpy as jnp
from jax import lax
from jax.experimental import pallas as pl
from jax.experimental.pallas import tpu as pltpu
from jax.experimental.pallas import tpu_sc as plsc

_INFO = plsc.get_sparse_core_info()
_NC = _INFO.num_cores
_NS = _INFO.num_subcores
_NW = _NC * _NS
_G = 128
_CPW = 4
_BPW = _CPW * _G


def _make_gather(batch, hist, vocab, dim):
    assert batch == _NW * _BPW and dim == 32
    n_rt = dim // 8
    mesh = plsc.VectorSubcoreMesh(core_axis_name="c", subcore_axis_name="s")

    @functools.partial(
        pl.kernel,
        out_type=jax.ShapeDtypeStruct((hist, n_rt, batch // _G, 8 * _G), jnp.float32),
        mesh=mesh,
        scratch_types=[
            pltpu.VMEM((2, _BPW), jnp.int32),
            pltpu.VMEM((2, _BPW, dim), jnp.float32),
            pltpu.VMEM((2, _CPW, n_rt, 8 * _G), jnp.float32),
            pltpu.VMEM((16, 16), jnp.int32),
            pltpu.VMEM((16, 16), jnp.int32),
            pltpu.SemaphoreType.DMA,
            pltpu.SemaphoreType.DMA,
            pltpu.SemaphoreType.DMA,
            pltpu.SemaphoreType.DMA,
        ],
        compiler_params=pltpu.CompilerParams(
            use_tc_tiling_on_sc=False,
            needs_layout_passes=False,
            disable_bounds_checks=True,
        ),
    )
    def gather_kernel(idx_hbm, table_hbm, out_hbm, idx_v, rows_v, tv, perm_v,
                      pofs_v, isem, gsem0, gsem1, osem):
        gsems = [gsem0, gsem1]
        c = lax.axis_index("c")
        s = lax.axis_index("s")
        wid = s * _NC + c
        b0 = wid * _BPW

        iota = lax.iota(jnp.int32, 16)
        for sh in range(16):
            p = lax.rem(iota + sh, 16)
            perm_v[sh] = p
            pofs_v[sh] = lax.rem(iota, 8) * 128 + p
        rowsel = lax.shift_right_logical(iota, 3)
        rowsel_dh = [rowsel, rowsel + 2]
        col_dh = [iota, iota + 16]

        def wait_idx(buf):
            pltpu.make_async_copy(
                idx_hbm.at[0, pl.ds(0, _BPW)], idx_v.at[buf], isem
            ).wait()

        def wait_rows(buf):
            pltpu.make_async_copy(
                table_hbm.at[pl.ds(0, _BPW)], rows_v.at[buf], gsems[buf]
            ).wait()

        def wait_out(buf):
            pltpu.make_async_copy(
                tv.at[buf], out_hbm.at[0, pl.ds(0, n_rt), pl.ds(0, _CPW)], osem
            ).wait()

        def fire_idx(h, buf):
            pltpu.async_copy(idx_hbm.at[h, pl.ds(b0, _BPW)], idx_v.at[buf], isem)

        def fire_gathers(buf):
            for j in range(_CPW):
                pltpu.async_copy(
                    table_hbm.at[idx_v.at[buf, pl.ds(j * _G, _G)]],
                    rows_v.at[buf, pl.ds(j * _G, _G)],
                    gsems[buf],
                )

        def transpose(buf):
            def s_body(sq, carry):
                for si in range(4):
                    sh = 4 * sq + si
                    perm = perm_v[sh]
                    pofs = pofs_v[sh]
                    for j in range(8):
                        pofs_j = pofs + (j * 16)
                        for cc in range(_CPW):
                            src_row = perm + (cc * _G + j * 16)
                            for dh in range(2):
                                vals = plsc.load_gather(
                                    rows_v.at[buf], [src_row, col_dh[dh]]
                                )
                                plsc.store_scatter(
                                    tv.at[buf, cc],
                                    [rowsel_dh[dh], pofs_j],
                                    vals,
                                )
                return carry

            lax.fori_loop(0, 4, s_body, 0)

        def fire_writes(h, buf):
            for cc in range(_CPW):
                pltpu.async_copy(
                    tv.at[buf, cc],
                    out_hbm.at[h, pl.ds(0, n_rt), wid * _CPW + cc],
                    osem,
                )

        fire_idx(0, 0)
        wait_idx(0)
        fire_gathers(0)
        fire_idx(1, 1)

        def step(h, b):
            ob = 1 - b

            @pl.when(h <= hist - 1)
            def _():
                wait_idx(b)
                fire_gathers(b)

            wait_rows(ob)

            @pl.when(h + 1 <= hist - 1)
            def _():
                fire_idx(h + 1, ob)

            @pl.when(h >= 3)
            def _():
                wait_out(ob)

            transpose(ob)
            fire_writes(h - 1, ob)

        def pair(t, carry):
            step(2 * t + 1, 1)
            step(2 * t + 2, 0)
            return carry

        lax.fori_loop(0, hist // 2, pair, 0)

        wait_out(0)
        wait_out(1)

    return gather_kernel


def kernel(input_tokens, table):
    batch, hist = input_tokens.shape
    vocab, dim = table.shape
    idx_t = input_tokens.T.astype(jnp.int32)
    out5 = _make_gather(batch, hist, vocab, dim)(idx_t, table)
    out = out5.reshape(hist, dim // 8, batch // _G, 8, _G)
    out = out.transpose(2, 4, 0, 1, 3)
    return out.reshape(batch, hist, dim)


# speedup vs baseline: 1.6325x; 1.0104x over previous
import functools

import jax
import jax.numpy as jnp
from jax import lax
from jax.experimental import pallas as pl
from jax.experimental.pallas import tpu as pltpu
from jax.experimental.pallas import tpu_sc as plsc

_INFO = plsc.get_sparse_core_info()
_NC = _INFO.num_cores
_NS = _INFO.num_subcores
_NW = _NC * _NS
_G = 128
_CPW = 4
_BPW = _CPW * _G


def _make_gather(batch, hist, vocab, dim):
    assert batch == _NW * _BPW and dim == 32
    n_rt = dim // 8
    mesh = plsc.VectorSubcoreMesh(core_axis_name="c", subcore_axis_name="s")

    @functools.partial(
        pl.kernel,
        out_type=jax.ShapeDtypeStruct((hist, n_rt, batch // _G, 8 * _G), jnp.float32),
        mesh=mesh,
        scratch_types=[
            pltpu.VMEM((2, _BPW), jnp.int32),
            pltpu.VMEM((2, _BPW, dim), jnp.float32),
            pltpu.VMEM((2, _CPW, n_rt, 8 * _G), jnp.float32),
            pltpu.VMEM((16, 16), jnp.int32),
            pltpu.VMEM((16, 16), jnp.int32),
            pltpu.SemaphoreType.DMA,
            pltpu.SemaphoreType.DMA,
            pltpu.SemaphoreType.DMA,
        ],
        compiler_params=pltpu.CompilerParams(
            use_tc_tiling_on_sc=False,
            needs_layout_passes=False,
            disable_bounds_checks=True,
        ),
    )
    def gather_kernel(idx_hbm, table_hbm, out_hbm, idx_v, rows_v, tv, perm_v,
                      pofs_v, isem, gsem, osem):
        c = lax.axis_index("c")
        s = lax.axis_index("s")
        wid = s * _NC + c
        b0 = wid * _BPW

        iota = lax.iota(jnp.int32, 16)
        for sh in range(16):
            p = lax.rem(iota + sh, 16)
            perm_v[sh] = p
            pofs_v[sh] = lax.rem(iota, 8) * 128 + p
        rowsel = lax.shift_right_logical(iota, 3)
        rowsel_dh = [rowsel, rowsel + 2]
        col_dh = [iota, iota + 16]

        def wait_idx(buf):
            pltpu.make_async_copy(
                idx_hbm.at[0, pl.ds(0, _BPW)], idx_v.at[buf], isem
            ).wait()

        def wait_rows(buf):
            pltpu.make_async_copy(
                table_hbm.at[pl.ds(0, _BPW)], rows_v.at[buf], gsem
            ).wait()

        def wait_out(buf):
            pltpu.make_async_copy(
                tv.at[buf], out_hbm.at[0, pl.ds(0, n_rt), pl.ds(0, _CPW)], osem
            ).wait()

        def fire_idx(h, buf):
            pltpu.async_copy(idx_hbm.at[h, pl.ds(b0, _BPW)], idx_v.at[buf], isem)

        def fire_gathers(buf):
            for j in range(_CPW):
                pltpu.async_copy(
                    table_hbm.at[idx_v.at[buf, pl.ds(j * _G, _G)]],
                    rows_v.at[buf, pl.ds(j * _G, _G)],
                    gsem,
                )

        def transpose(buf):
            def s_body(sq, carry):
                for si in range(4):
                    sh = 4 * sq + si
                    perm = perm_v[sh]
                    pofs = pofs_v[sh]
                    for j in range(8):
                        pofs_j = pofs + (j * 16)
                        for cc in range(_CPW):
                            src_row = perm + (cc * _G + j * 16)
                            for dh in range(2):
                                vals = plsc.load_gather(
                                    rows_v.at[buf], [src_row, col_dh[dh]]
                                )
                                plsc.store_scatter(
                                    tv.at[buf, cc],
                                    [rowsel_dh[dh], pofs_j],
                                    vals,
                                )
                return carry

            lax.fori_loop(0, 4, s_body, 0)

        def fire_writes(h, buf):
            for cc in range(_CPW):
                pltpu.async_copy(
                    tv.at[buf, cc],
                    out_hbm.at[h, pl.ds(0, n_rt), wid * _CPW + cc],
                    osem,
                )

        fire_idx(0, 0)
        wait_idx(0)
        fire_gathers(0)
        fire_idx(1, 1)

        def step(h, b):
            ob = 1 - b
            wait_rows(ob)

            @pl.when(h + 1 <= hist - 1)
            def _():
                fire_idx(h + 1, ob)

            @pl.when(h <= hist - 1)
            def _():
                wait_idx(b)
                fire_gathers(b)

            @pl.when(h >= 3)
            def _():
                wait_out(ob)

            transpose(ob)
            fire_writes(h - 1, ob)

        def pair(t, carry):
            step(2 * t + 1, 1)
            step(2 * t + 2, 0)
            return carry

        lax.fori_loop(0, hist // 2, pair, 0)

        wait_out(0)
        wait_out(1)

    return gather_kernel


def kernel(input_tokens, table):
    batch, hist = input_tokens.shape
    vocab, dim = table.shape
    idx_t = input_tokens.T.astype(jnp.int32)
    out5 = _make_gather(batch, hist, vocab, dim)(idx_t, table)
    out = out5.reshape(hist, dim // 8, batch // _G, 8, _G)
    out = out.transpose(2, 4, 0, 1, 3)
    return out.reshape(batch, hist, dim)
